# Initial kernel scaffold; baseline (speedup 1.0000x reference)
import jax
import jax.numpy as jnp
from jax.experimental import pallas as pl


def kernel(x, abs_x, deg, idx, Wq, Wk, Wv):
    raise NotImplementedError("write your pallas kernel here")


# speedup vs baseline: 1.0344x; 1.0344x over previous
import jax
import jax.numpy as jnp
from jax import lax
from jax.experimental import pallas as pl

B, CIN, N, K, CO = 2, 128, 512, 20, 64
NEG = -1e30


def _body(x_ref, ax_ref, it_ref, mask_ref, wq_ref, wk_ref, wv_ref, o_ref):
    wq = wq_ref[...]
    wk = wk_ref[...]
    wv = wv_ref[...]

    q = []
    att = []
    k0 = None
    for k in range(K):
        xk = x_ref[0, k]
        axk = ax_ref[0, k]
        qk = jnp.dot(axk, wq, preferred_element_type=jnp.float32)
        kk = jnp.dot(xk, wk, preferred_element_type=jnp.float32)
        if k == 0:
            k0 = kk
        q.append(qk)
        att.append(qk * kk)
    m = att[0]
    for k in range(1, K):
        m = jnp.maximum(m, att[k])
    e = [jnp.exp(att[k] - m) for k in range(K)]
    s = e[0]
    for k in range(1, K):
        s = s + e[k]
    out = [e[k] / s for k in range(K)]

    iota_m = lax.broadcasted_iota(jnp.int32, (N, N), 0)
    score = jnp.zeros((N, CO), dtype=jnp.float32)
    out_f = jnp.zeros((N, CO), dtype=jnp.float32)
    v0 = None
    for k in range(K):
        xk = x_ref[0, k]
        vk = jnp.dot(xk, wv, preferred_element_type=jnp.float32)
        if k == 0:
            v0 = vk
        out_f = out_f + out[k] * vk
        idx_row = it_ref[0, k : k + 1, :]
        oh = (iota_m == idx_row).astype(jnp.float32)
        outm = out[k] * mask_ref[0, k]
        score = score + jnp.dot(oh, outm, preferred_element_type=jnp.float32,
                                precision=lax.Precision.HIGHEST)

    row_id = lax.broadcasted_iota(jnp.int32, (N, CO), 0)
    sw = score
    k_sal = []
    v_sal = []
    for _ in range(20):
        cm = jnp.max(sw, axis=0, keepdims=True)
        cand = jnp.where(sw == cm, row_id, N)
        sel = jnp.min(cand, axis=0, keepdims=True)
        hit = row_id == sel
        k_sal.append(jnp.sum(jnp.where(hit, k0, 0.0), axis=0, keepdims=True))
        v_sal.append(jnp.sum(jnp.where(hit, v0, 0.0), axis=0, keepdims=True))
        sw = jnp.where(hit, NEG, sw)

    t = [q[k] * k_sal[k] for k in range(K)]
    m2 = t[0]
    for k in range(1, K):
        m2 = jnp.maximum(m2, t[k])
    e2 = [jnp.exp(t[k] - m2) for k in range(K)]
    s2 = e2[0]
    for k in range(1, K):
        s2 = s2 + e2[k]
    out_all = jnp.zeros((N, CO), dtype=jnp.float32)
    for k in range(K):
        out_all = out_all + (e2[k] / s2) * v_sal[k]

    o_ref[0] = out_f + out_all


def _keep_mask(idx_last):
    b_i = jnp.arange(B, dtype=jnp.int32)[:, None, None, None]
    c_i = jnp.arange(CO, dtype=jnp.int32)[None, :, None, None]
    n_i = jnp.arange(N, dtype=jnp.int32)[None, None, :, None]
    keys = (((b_i * CO + c_i) * N + n_i) * N + idx_last[:, None, :, :]).reshape(-1)
    pos = jnp.arange(keys.shape[0], dtype=jnp.float32)
    skeys, spos = lax.sort((keys, pos), num_keys=1, is_stable=False)
    run_end = jnp.concatenate([skeys[1:] != skeys[:-1],
                               jnp.ones((1,), jnp.bool_)])
    flags = jnp.where(run_end, 1.0, 0.0)
    mask = jnp.zeros_like(flags).at[spos.astype(jnp.int32)].set(
        flags, unique_indices=True)
    return mask.reshape(B, CO, N, K)


@jax.jit
def kernel(x, abs_x, deg, idx, Wq, Wk, Wv):
    del deg
    xt = x.reshape(B, CIN, N * K).transpose(0, 2, 1).reshape(B, N, K, CIN)
    xt = xt.transpose(0, 2, 1, 3)
    axt = abs_x.reshape(B, CIN // 2, N * K).transpose(0, 2, 1)
    axt = axt.reshape(B, N, K, CIN // 2).transpose(0, 2, 1, 3)
    ia = idx[:, :, :, -1]
    it = ia.transpose(0, 2, 1)
    mask = _keep_mask(ia).transpose(0, 3, 2, 1)

    out = pl.pallas_call(
        _body,
        grid=(B,),
        in_specs=[
            pl.BlockSpec((1, K, N, CIN), lambda b: (b, 0, 0, 0)),
            pl.BlockSpec((1, K, N, CIN // 2), lambda b: (b, 0, 0, 0)),
            pl.BlockSpec((1, K, N), lambda b: (b, 0, 0)),
            pl.BlockSpec((1, K, N, CO), lambda b: (b, 0, 0, 0)),
            pl.BlockSpec((CIN // 2, CO), lambda b: (0, 0)),
            pl.BlockSpec((CIN, CO), lambda b: (0, 0)),
            pl.BlockSpec((CIN, CO), lambda b: (0, 0)),
        ],
        out_specs=pl.BlockSpec((1, N, CO), lambda b: (b, 0, 0)),
        out_shape=jax.ShapeDtypeStruct((B, N, CO), jnp.float32),
    )(xt, axt, it, mask, Wq.T, Wk.T, Wv.T)
    return out.transpose(0, 2, 1)[..., None]
